# baseline (speedup 1.0000x reference)
import functools

import jax
import jax.numpy as jnp
from jax import lax
from jax.experimental import pallas as pl
from jax.experimental.pallas import tpu as pltpu
from jax.experimental.pallas import tpu_sc as plsc

DIM = 64
SCALE = 8.0
NC, NS, LANES = 2, 16, 16
NW = NC * NS
IDXROW = 128


def kernel(tokens, table):
    B, L = tokens.shape
    Bt = B * L
    per_w = Bt // NW
    nrows = per_w // IDXROW
    G = 2
    CH = G * IDXROW
    nch = nrows // G

    idx = tokens.astype(jnp.int32).reshape(NW, nrows, IDXROW)

    mesh = plsc.VectorSubcoreMesh(core_axis_name="c", subcore_axis_name="s")

    @functools.partial(
        pl.kernel,
        mesh=mesh,
        compiler_params=pltpu.CompilerParams(use_tc_tiling_on_sc=False),
        out_type=jax.ShapeDtypeStruct((Bt, DIM), jnp.float32),
        scratch_types=[
            pltpu.VMEM((nrows, IDXROW), jnp.int32),
            pltpu.VMEM((CH, DIM), jnp.float32),
            pltpu.SemaphoreType.DMA,
        ],
    )
    def emb_kernel(tok_hbm, tab_hbm, out_hbm, idx_v, rows_v, gsem):
        wid = lax.axis_index("s") * NC + lax.axis_index("c")
        base = wid * per_w
        pltpu.sync_copy(tok_hbm.at[wid], idx_v)

        @pl.loop(0, nch)
        def chunk_loop(c):
            copies = [
                pltpu.async_copy(
                    tab_hbm.at[idx_v.at[c * G + g]],
                    rows_v.at[pl.ds(g * IDXROW, IDXROW)],
                    gsem,
                )
                for g in range(G)
            ]
            for cp in copies:
                cp.wait()

            @pl.loop(0, CH)
            def scale_loop(r):
                for q in range(DIM // LANES):
                    sl = pl.ds(q * LANES, LANES)
                    rows_v[r, sl] = rows_v[r, sl] * SCALE

            pltpu.sync_copy(rows_v, out_hbm.at[pl.ds(base + c * CH, CH)])

    out = emb_kernel(idx, table)
    return out.reshape(B, L, DIM)


# speedup vs baseline: 1.1481x; 1.1481x over previous
import functools

import jax
import jax.numpy as jnp
from jax import lax
from jax.experimental import pallas as pl
from jax.experimental.pallas import tpu as pltpu
from jax.experimental.pallas import tpu_sc as plsc

DIM = 64
SCALE = 8.0
NC, NS, LANES = 2, 16, 16
NW = NC * NS
IDXROW = 128
NBUF = 4
G = 2
CH = G * IDXROW


def kernel(tokens, table):
    B, L = tokens.shape
    Bt = B * L
    per_w = Bt // NW
    nrows = per_w // IDXROW
    nch = nrows // G
    nt = nch // NBUF

    idx = tokens.astype(jnp.int32).reshape(NW, nrows, IDXROW)

    mesh = plsc.VectorSubcoreMesh(core_axis_name="c", subcore_axis_name="s")

    @functools.partial(
        pl.kernel,
        mesh=mesh,
        compiler_params=pltpu.CompilerParams(use_tc_tiling_on_sc=False),
        out_type=jax.ShapeDtypeStruct((Bt, DIM), jnp.float32),
        scratch_types=[
            pltpu.VMEM((nrows, IDXROW), jnp.int32),
            pltpu.VMEM((NBUF, CH, DIM), jnp.float32),
        ]
        + [pltpu.SemaphoreType.DMA] * (2 * NBUF),
    )
    def emb_kernel(tok_hbm, tab_hbm, out_hbm, idx_v, rows_v, *sems):
        gsem = sems[:NBUF]
        osem = sems[NBUF:]
        wid = lax.axis_index("s") * NC + lax.axis_index("c")
        base = wid * per_w
        pltpu.sync_copy(tok_hbm.at[wid], idx_v)

        def issue_gathers(c, b):
            for g in range(G):
                pltpu.async_copy(
                    tab_hbm.at[idx_v.at[c * G + g]],
                    rows_v.at[b].at[pl.ds(g * IDXROW, IDXROW)],
                    gsem[b],
                )

        def drain_gathers(c, b):
            for g in range(G):
                pltpu.make_async_copy(
                    tab_hbm.at[idx_v.at[c * G + g]],
                    rows_v.at[b].at[pl.ds(g * IDXROW, IDXROW)],
                    gsem[b],
                ).wait()

        def store(c, b):
            return pltpu.make_async_copy(
                rows_v.at[b], out_hbm.at[pl.ds(base + c * CH, CH)], osem[b]
            )

        issue_gathers(0, 0)
        issue_gathers(1, 1)

        @pl.loop(0, nt)
        def outer(t):
            for b in range(NBUF):
                c = t * NBUF + b
                nb = (b + 2) % NBUF
                if b < 2:
                    @pl.when(t > 0)
                    def _():
                        store(c - 2, nb).wait()

                    issue_gathers(c + 2, nb)
                else:
                    @pl.when(t < nt - 1)
                    def _():
                        store(c - 2, nb).wait()
                        issue_gathers(c + 2, nb)

                drain_gathers(c, b)

                @plsc.parallel_loop(0, CH, 1, unroll=8)
                def scale_loop(r):
                    for q in range(DIM // LANES):
                        sl = pl.ds(q * LANES, LANES)
                        rows_v[b, r, sl] = rows_v[b, r, sl] * SCALE

                store(c, b).start()

        for b in range(NBUF):
            store(nch - NBUF + b, b).wait()

    out = emb_kernel(idx, table)
    return out.reshape(B, L, DIM)
